# baseline (speedup 1.0000x reference)
import functools

import jax
import jax.numpy as jnp
from jax.experimental import pallas as pl

B = 32
N_IN = 3445
N_OUT = 6890
C = 128
SEQ = 9
OUT_C = 128

MM_BLOCK_M = 512


def _mm_kernel(sp_ref, w_ref, b_ref, o_ref):
    acc = jnp.dot(sp_ref[...], w_ref[...], preferred_element_type=jnp.float32)
    o_ref[...] = jnp.maximum(acc + b_ref[...], 0.0)


@functools.partial(jax.jit, static_argnums=())
def _matmul_relu(sp, wt, bias):
    m = sp.shape[0]
    grid = (pl.cdiv(m, MM_BLOCK_M),)
    return pl.pallas_call(
        _mm_kernel,
        grid=grid,
        in_specs=[
            pl.BlockSpec((MM_BLOCK_M, C * SEQ), lambda i: (i, 0)),
            pl.BlockSpec((C * SEQ, OUT_C), lambda i: (0, 0)),
            pl.BlockSpec((1, OUT_C), lambda i: (0, 0)),
        ],
        out_specs=pl.BlockSpec((MM_BLOCK_M, OUT_C), lambda i: (i, 0)),
        out_shape=jax.ShapeDtypeStruct((m, OUT_C), jnp.float32),
    )(sp, wt, bias)


def kernel(x, up_row, up_col, up_val, indices, W, b):
    gathered = jnp.take(x, up_col, axis=1) * up_val[None, :, None]
    pooled = jnp.zeros((B, N_OUT, C), jnp.float32).at[:, up_row, :].add(gathered)
    pooled_bf = pooled.astype(jnp.bfloat16)
    sp = jnp.take(pooled_bf, indices.reshape(-1), axis=1)
    sp = sp.reshape(B * N_OUT, SEQ * C)
    wt = W.T.astype(jnp.bfloat16)
    out = _matmul_relu(sp, wt, b.reshape(1, OUT_C))
    return out.reshape(B, N_OUT, OUT_C)
